# baseline (speedup 1.0000x reference)
import functools

import jax
import jax.numpy as jnp
from jax import lax
from jax.experimental import pallas as pl
from jax.experimental.pallas import tpu as pltpu
from jax.experimental.pallas import tpu_sc as plsc

NC = 2
NS = 16
L = 16
RADIX = 256
M = 4
SV = 2048
NV = NS * M


@functools.partial(jax.jit, static_argnums=(1,))
def _rank_both(both, n):
  C = n // NS
  S = M * SV
  nround = C // S
  mesh = plsc.VectorSubcoreMesh(
      core_axis_name="c", subcore_axis_name="s", num_cores=NC,
      num_subcores=NS)

  def body(both_hbm, ranks_hbm, ka, va, kb, vb,
           hist, hist1d, allhist, counter,
           fbuf, keyb, vals, posb, shist, sem):

    cid = lax.axis_index("c")
    sid = lax.axis_index("s")
    base0 = cid * n
    lanes = lax.iota(jnp.int32, L)
    ones = jnp.ones((L,), jnp.int32)
    zeros = jnp.zeros((L,), jnp.int32)

    wbase = sid * C

    def stage_load(rnd, src_k, src_v, with_vals):
      start = base0 + wbase + rnd * S
      pltpu.sync_copy(src_k.at[pl.ds(start, S)], keyb)
      if with_vals:
        pltpu.sync_copy(src_v.at[pl.ds(start, S)], vals)

    def digit_of(k, shift):
      return lax.bitwise_and(lax.shift_right_logical(k, shift),
                             jnp.int32(RADIX - 1))

    lg_cv = (C // M).bit_length() - 1
    lg_sv = SV.bit_length() - 1
    lg_s = S.bit_length() - 1
    lg_c = C.bit_length() - 1
    lg_m = M.bit_length() - 1
    nr_mask = nround - 1

    def sigma(q):
      vid = lax.shift_right_logical(q, lg_cv)
      rnd = lax.bitwise_and(lax.shift_right_logical(q, lg_sv),
                            jnp.int32(nr_mask))
      off = lax.bitwise_and(q, jnp.int32(SV - 1))
      sidq = lax.shift_right_logical(vid, lg_m)
      mq = lax.bitwise_and(vid, jnp.int32(M - 1))
      return (lax.shift_left(sidq, lg_c) + lax.shift_left(rnd, lg_s)
              + lax.shift_left(mq, lg_sv) + off)

    def run_pass(p, src_k, src_v, dst_k, dst_v, last_pass):
      shift = 8 * p

      def zrow(i, c):
        hist[pl.ds(i * L, L)] = zeros
        return c
      lax.fori_loop(0, (M * L * RADIX) // L, zrow, 0)

      for rnd in range(nround):
        if p == 0:
          pltpu.sync_copy(both_hbm.at[pl.ds(base0 + wbase + rnd * S, S)],
                          fbuf)

          def cvt(j, c):
            for m in range(M):
              sl = pl.ds(m * SV + j * L, L)
              b = lax.bitcast_convert_type(fbuf[sl], jnp.int32)
              sgn = lax.shift_right_arithmetic(b, 31)
              keyb[sl] = lax.bitwise_xor(
                  b, lax.bitwise_or(sgn, jnp.int32(-(2**31))))
            return c
          lax.fori_loop(0, SV // L, cvt, 0)
        else:
          stage_load(rnd, src_k, src_v, False)

        def hb(j, c):
          for m in range(M):
            d = digit_of(keyb[pl.ds(m * SV + j * L, L)], shift)
            idx = (m * L) * RADIX + lanes * RADIX + d
            plsc.addupdate_scatter(hist, [idx], ones)
          return c
        lax.fori_loop(0, SV // L, hb, 0)

      plsc.subcore_barrier()

      for m in range(M):
        def red(jv, c, m=m):
          def rsum(l, acc, m=m):
            return acc + hist[pl.ds((m * L + l) * RADIX + jv * L, L)]
          hist1d[pl.ds(jv * L, L)] = lax.fori_loop(0, L, rsum, zeros)
          return c
        lax.fori_loop(0, RADIX // L, red, 0)
        pltpu.sync_copy(
            hist1d, shist.at[pl.ds((sid * M + m) * RADIX, RADIX)])

      plsc.subcore_barrier()
      pltpu.sync_copy(shist, allhist)
      plsc.subcore_barrier()

      for m in range(M):
        vid = sid * M + m

        def scan_body(jv, carry, m=m, vid=vid):
          sl = pl.ds(jv * L, L)

          def acc_body(r, accs, vid=vid):
            tot, below = accs
            row = allhist[pl.ds(r * RADIX + jv * L, L)]
            sel = jnp.broadcast_to(r, (L,)) < jnp.broadcast_to(vid, (L,))
            return tot + row, below + jnp.where(sel, row, zeros)
          tot, below = lax.fori_loop(0, NV, acc_body, (zeros, zeros))
          excl = plsc.cumsum(tot) - tot + jnp.broadcast_to(carry, (L,))
          counter[pl.ds(m * RADIX + jv * L, L)] = excl + below
          return carry + jnp.sum(tot)
        lax.fori_loop(0, RADIX // L, scan_body, jnp.int32(0))

      for rnd in range(nround):
        if p == 0:
          pltpu.sync_copy(both_hbm.at[pl.ds(base0 + wbase + rnd * S, S)],
                          fbuf)

          def cvt2(j, c):
            for m in range(M):
              slm = pl.ds(m * SV + j * L, L)
              b = lax.bitcast_convert_type(fbuf[slm], jnp.int32)
              sgn = lax.shift_right_arithmetic(b, 31)
              keyb[slm] = lax.bitwise_xor(
                  b, lax.bitwise_or(sgn, jnp.int32(-(2**31))))
              vals[slm] = (wbase + rnd * S + m * SV + j * L) + lanes
            return c
          lax.fori_loop(0, SV // L, cvt2, 0)
        else:
          stage_load(rnd, src_k, src_v, True)

        def pb(j, c):
          for m in range(M):
            slm = pl.ds(m * SV + j * L, L)
            d = digit_of(keyb[slm], shift)
            cnt, lastm = plsc.scan_count(d)
            dc = jnp.broadcast_to(jnp.int32(m * RADIX), (L,)) + d
            prior = plsc.load_gather(counter, [dc])
            pos = prior + cnt - 1
            plsc.store_scatter(counter, [dc], prior + cnt, mask=lastm)
            if last_pass:
              fbuf[slm] = lax.convert_element_type(pos, jnp.float32)
              vals[slm] = vals[slm] + jnp.broadcast_to(base0, (L,))
            else:
              posb[slm] = sigma(pos) + jnp.broadcast_to(base0, (L,))
          return c
        lax.fori_loop(0, SV // L, pb, 0)

        if last_pass:
          pltpu.async_copy(fbuf, ranks_hbm.at[vals], sem).wait()
        else:
          c1 = pltpu.async_copy(keyb, dst_k.at[posb], sem)
          c2 = pltpu.async_copy(vals, dst_v.at[posb], sem)
          c1.wait()
          c2.wait()
      plsc.subcore_barrier()

    run_pass(0, None, None, ka, va, False)
    run_pass(1, ka, va, kb, vb, False)
    run_pass(2, kb, vb, ka, va, False)
    run_pass(3, ka, va, None, None, True)

  call = pl.kernel(
      body,
      out_type=[
          jax.ShapeDtypeStruct((2 * n,), jnp.float32),
          jax.ShapeDtypeStruct((2 * n,), jnp.int32),
          jax.ShapeDtypeStruct((2 * n,), jnp.int32),
          jax.ShapeDtypeStruct((2 * n,), jnp.int32),
          jax.ShapeDtypeStruct((2 * n,), jnp.int32),
      ],
      mesh=mesh,
      compiler_params=pltpu.CompilerParams(needs_layout_passes=False),
      scratch_types=(
          [
              pltpu.VMEM((M * L * RADIX,), jnp.int32),
              pltpu.VMEM((RADIX,), jnp.int32),
              pltpu.VMEM((NV * RADIX,), jnp.int32),
              pltpu.VMEM((M * RADIX,), jnp.int32),
          ]
          + [
              pltpu.VMEM((M * SV,), jnp.float32),
              pltpu.VMEM((M * SV,), jnp.int32),
              pltpu.VMEM((M * SV,), jnp.int32),
              pltpu.VMEM((M * SV,), jnp.int32),
          ]
          + [
              pltpu.VMEM_SHARED((NV * RADIX,), jnp.int32),
              pltpu.SemaphoreType.DMA,
          ]
      ),
  )
  return call(both)[0]


@functools.partial(jax.jit, static_argnums=(2,))
def _spearman_from_ranks(rx, ry, n):
  rows, cols = rx.shape
  grid = 8
  br = rows // grid
  mean = (n - 1) * 0.5
  sig2 = float((n * (n * n - 1)) // 12)

  def body(rx_ref, ry_ref, o_ref, acc_ref):
    i = pl.program_id(0)

    @pl.when(i == 0)
    def _():
      acc_ref[...] = jnp.zeros_like(acc_ref)

    x = rx_ref[...] - jnp.float32(mean)
    y = ry_ref[...] - jnp.float32(mean)
    acc_ref[...] += x * y

    @pl.when(i == pl.num_programs(0) - 1)
    def _():
      a = acc_ref[...]
      c = cols
      while c > 128:
        c //= 2
        a = a[:, :c] + a[:, c:]
      r = br
      while r > 8:
        r //= 2
        a = a[:r, :] + a[r:, :]
      s = jnp.sum(a)
      o_ref[...] = jnp.full(o_ref.shape,
                            jnp.clip(s * jnp.float32(1.0 / sig2), -1.0, 1.0),
                            jnp.float32)

  out = pl.pallas_call(
      body,
      grid=(grid,),
      in_specs=[
          pl.BlockSpec((br, cols), lambda i: (i, 0)),
          pl.BlockSpec((br, cols), lambda i: (i, 0)),
      ],
      out_specs=pl.BlockSpec((8, 128), lambda i: (0, 0)),
      out_shape=jax.ShapeDtypeStruct((8, 128), jnp.float32),
      scratch_shapes=[pltpu.VMEM((br, cols), jnp.float32)],
  )(rx, ry)
  return out[0, 0]


def kernel(preds, targets):
  preds = jnp.squeeze(preds)
  targets = jnp.squeeze(targets)
  n = preds.shape[0]
  both = jnp.concatenate([preds, targets])
  ranks = _rank_both(both, n)
  cols = 1024
  rx = ranks[:n].reshape(n // cols, cols)
  ry = ranks[n:].reshape(n // cols, cols)
  return _spearman_from_ranks(rx, ry, n)
